# baseline (speedup 1.0000x reference)
import functools

import jax
import jax.numpy as jnp
from jax import lax
from jax.experimental import pallas as pl
from jax.experimental.pallas import tpu as pltpu
from jax.experimental.pallas import tpu_sc as plsc

_VOCAB = 100000
_HID = 128

_NC = 2
_NS = 16
_NW = _NC * _NS

_ROWS_BLK = 20000

_CHUNK = 400

_NSLOT = 2


def _linear_body(t_ref, w_ref, b_ref, o_ref):
    o_ref[...] = lax.dot_general(
        t_ref[...], w_ref[...],
        (((1,), (1,)), ((), ())),
        preferred_element_type=jnp.float32,
    ) + b_ref[...]


def _transform_table(table, W, b):
    grid = _VOCAB // _ROWS_BLK
    return pl.pallas_call(
        _linear_body,
        grid=(grid,),
        in_specs=[
            pl.BlockSpec((_ROWS_BLK, _HID), lambda i: (i, 0)),
            pl.BlockSpec((_HID, _HID), lambda i: (0, 0)),
            pl.BlockSpec((1, _HID), lambda i: (0, 0)),
        ],
        out_specs=pl.BlockSpec((_ROWS_BLK, _HID), lambda i: (i, 0)),
        out_shape=jax.ShapeDtypeStruct((_VOCAB, _HID), jnp.float32),
    )(table, W, b.reshape(1, _HID))


def _make_gather(n_tokens):
    b_per_w = n_tokens // _NW
    n_chunks = b_per_w // _CHUNK
    mesh = plsc.VectorSubcoreMesh(core_axis_name="c", subcore_axis_name="s")

    @functools.partial(
        pl.kernel,
        mesh=mesh,
        out_type=jax.ShapeDtypeStruct((n_tokens, _HID), jnp.float32),
        scratch_types=(
            [pltpu.VMEM((b_per_w,), jnp.int32)]
            + [pltpu.VMEM((_CHUNK, _HID), jnp.float32) for _ in range(_NSLOT)]
            + [pltpu.SemaphoreType.DMA for _ in range(2 * _NSLOT)]
        ),
    )
    def gather_k(table_hbm, idx_hbm, out_hbm, idx_v, *scratch):
        rows = scratch[:_NSLOT]
        gsem = scratch[_NSLOT:2 * _NSLOT]
        osem = scratch[2 * _NSLOT:]
        wid = lax.axis_index("s") * _NC + lax.axis_index("c")
        base = wid * b_per_w
        pltpu.sync_copy(idx_hbm.at[pl.ds(base, b_per_w)], idx_v)

        def fire_gather(c, s):
            return pltpu.async_copy(
                table_hbm.at[idx_v.at[pl.ds(c * _CHUNK, _CHUNK)]], rows[s],
                gsem[s])

        def fire_write(c, s):
            return pltpu.async_copy(
                rows[s], out_hbm.at[pl.ds(base + c * _CHUNK, _CHUNK)], osem[s])

        def drain_write(s):
            pltpu.make_async_copy(
                rows[s], out_hbm.at[pl.ds(base, _CHUNK)], osem[s]).wait()

        def body(p, _):
            for s in range(_NSLOT):
                @pl.when(p > 0)
                def _(s=s):
                    drain_write(s)
                fire_gather(p * _NSLOT + s, s)
            for s in range(_NSLOT):
                pltpu.make_async_copy(
                    table_hbm.at[idx_v.at[pl.ds(0, _CHUNK)]], rows[s],
                    gsem[s]).wait()
                fire_write(p * _NSLOT + s, s)
            return 0

        lax.fori_loop(0, n_chunks // _NSLOT, body, 0)
        for s in range(_NSLOT):
            drain_write(s)

    return gather_k


def kernel(input_ids, table, W, b):
    B, L = input_ids.shape
    n_tokens = B * L
    table2 = _transform_table(table, W, b)
    flat_idx = input_ids.reshape(n_tokens).astype(jnp.int32)
    out = _make_gather(n_tokens)(table2, flat_idx)
    return out.reshape(B, L, _HID)


# speedup vs baseline: 1.0098x; 1.0098x over previous
import functools

import jax
import jax.numpy as jnp
from jax import lax
from jax.experimental import pallas as pl
from jax.experimental.pallas import tpu as pltpu
from jax.experimental.pallas import tpu_sc as plsc

_VOCAB = 100000
_HID = 128

_NC = 2
_NS = 16
_NW = _NC * _NS

_ROWS_BLK = 20000

_CHUNK = 160

_NSLOT = 5


def _linear_body(t_ref, w_ref, b_ref, o_ref):
    o_ref[...] = lax.dot_general(
        t_ref[...], w_ref[...],
        (((1,), (1,)), ((), ())),
        preferred_element_type=jnp.float32,
    ) + b_ref[...]


def _transform_table(table, W, b):
    grid = _VOCAB // _ROWS_BLK
    return pl.pallas_call(
        _linear_body,
        grid=(grid,),
        in_specs=[
            pl.BlockSpec((_ROWS_BLK, _HID), lambda i: (i, 0)),
            pl.BlockSpec((_HID, _HID), lambda i: (0, 0)),
            pl.BlockSpec((1, _HID), lambda i: (0, 0)),
        ],
        out_specs=pl.BlockSpec((_ROWS_BLK, _HID), lambda i: (i, 0)),
        out_shape=jax.ShapeDtypeStruct((_VOCAB, _HID), jnp.float32),
    )(table, W, b.reshape(1, _HID))


def _make_gather(n_tokens):
    b_per_w = n_tokens // _NW
    n_chunks = b_per_w // _CHUNK
    mesh = plsc.VectorSubcoreMesh(core_axis_name="c", subcore_axis_name="s")

    @functools.partial(
        pl.kernel,
        mesh=mesh,
        out_type=jax.ShapeDtypeStruct((n_tokens, _HID), jnp.float32),
        scratch_types=(
            [pltpu.VMEM((b_per_w,), jnp.int32)]
            + [pltpu.VMEM((_CHUNK, _HID), jnp.float32) for _ in range(_NSLOT)]
            + [pltpu.SemaphoreType.DMA for _ in range(2 * _NSLOT)]
        ),
    )
    def gather_k(table_hbm, idx_hbm, out_hbm, idx_v, *scratch):
        rows = scratch[:_NSLOT]
        gsem = scratch[_NSLOT:2 * _NSLOT]
        osem = scratch[2 * _NSLOT:]
        wid = lax.axis_index("s") * _NC + lax.axis_index("c")
        base = wid * b_per_w
        pltpu.sync_copy(idx_hbm.at[pl.ds(base, b_per_w)], idx_v)

        def fire_gather(c, s):
            return pltpu.async_copy(
                table_hbm.at[idx_v.at[pl.ds(c * _CHUNK, _CHUNK)]], rows[s],
                gsem[s])

        def fire_write(c, s):
            return pltpu.async_copy(
                rows[s], out_hbm.at[pl.ds(base + c * _CHUNK, _CHUNK)], osem[s])

        def drain_write(s):
            pltpu.make_async_copy(
                rows[s], out_hbm.at[pl.ds(base, _CHUNK)], osem[s]).wait()

        def body(p, _):
            for s in range(_NSLOT):
                @pl.when(p > 0)
                def _(s=s):
                    drain_write(s)
                fire_gather(p * _NSLOT + s, s)
            for s in range(_NSLOT):
                pltpu.make_async_copy(
                    table_hbm.at[idx_v.at[pl.ds(0, _CHUNK)]], rows[s],
                    gsem[s]).wait()
                fire_write(p * _NSLOT + s, s)
            return 0

        lax.fori_loop(0, n_chunks // _NSLOT, body, 0)
        for s in range(_NSLOT):
            drain_write(s)

    return gather_k


def kernel(input_ids, table, W, b):
    B, L = input_ids.shape
    n_tokens = B * L
    table2 = _transform_table(table, W, b)
    flat_idx = input_ids.reshape(n_tokens).astype(jnp.int32)
    out = _make_gather(n_tokens)(table2, flat_idx)
    return out.reshape(B, L, _HID)


# speedup vs baseline: 1.0177x; 1.0078x over previous
import functools

import jax
import jax.numpy as jnp
from jax import lax
from jax.experimental import pallas as pl
from jax.experimental.pallas import tpu as pltpu
from jax.experimental.pallas import tpu_sc as plsc

_VOCAB = 100000
_HID = 128

_NC = 2
_NS = 16
_NW = _NC * _NS

_ROWS_BLK = 20000

_CHUNK = 200

_NSLOT = 4


def _linear_body(t_ref, w_ref, b_ref, o_ref):
    o_ref[...] = lax.dot_general(
        t_ref[...], w_ref[...],
        (((1,), (1,)), ((), ())),
        preferred_element_type=jnp.float32,
    ) + b_ref[...]


def _transform_table(table, W, b):
    grid = _VOCAB // _ROWS_BLK
    return pl.pallas_call(
        _linear_body,
        grid=(grid,),
        in_specs=[
            pl.BlockSpec((_ROWS_BLK, _HID), lambda i: (i, 0)),
            pl.BlockSpec((_HID, _HID), lambda i: (0, 0)),
            pl.BlockSpec((1, _HID), lambda i: (0, 0)),
        ],
        out_specs=pl.BlockSpec((_ROWS_BLK, _HID), lambda i: (i, 0)),
        out_shape=jax.ShapeDtypeStruct((_VOCAB, _HID), jnp.float32),
    )(table, W, b.reshape(1, _HID))


def _make_gather(n_tokens):
    b_per_w = n_tokens // _NW
    n_chunks = b_per_w // _CHUNK
    mesh = plsc.VectorSubcoreMesh(core_axis_name="c", subcore_axis_name="s")

    @functools.partial(
        pl.kernel,
        mesh=mesh,
        out_type=jax.ShapeDtypeStruct((n_tokens, _HID), jnp.float32),
        scratch_types=(
            [pltpu.VMEM((b_per_w,), jnp.int32)]
            + [pltpu.VMEM((_CHUNK, _HID), jnp.float32) for _ in range(_NSLOT)]
            + [pltpu.SemaphoreType.DMA for _ in range(2 * _NSLOT)]
        ),
    )
    def gather_k(table_hbm, idx_hbm, out_hbm, idx_v, *scratch):
        rows = scratch[:_NSLOT]
        gsem = scratch[_NSLOT:2 * _NSLOT]
        osem = scratch[2 * _NSLOT:]
        wid = lax.axis_index("s") * _NC + lax.axis_index("c")
        base = wid * b_per_w
        pltpu.sync_copy(idx_hbm.at[pl.ds(base, b_per_w)], idx_v)

        def fire_gather(c, s):
            return pltpu.async_copy(
                table_hbm.at[idx_v.at[pl.ds(c * _CHUNK, _CHUNK)]], rows[s],
                gsem[s])

        def fire_write(c, s):
            return pltpu.async_copy(
                rows[s], out_hbm.at[pl.ds(base + c * _CHUNK, _CHUNK)], osem[s])

        def drain_write(s):
            pltpu.make_async_copy(
                rows[s], out_hbm.at[pl.ds(base, _CHUNK)], osem[s]).wait()

        def body(p, _):
            for s in range(_NSLOT):
                @pl.when(p > 0)
                def _(s=s):
                    drain_write(s)
                fire_gather(p * _NSLOT + s, s)
            for s in range(_NSLOT):
                pltpu.make_async_copy(
                    table_hbm.at[idx_v.at[pl.ds(0, _CHUNK)]], rows[s],
                    gsem[s]).wait()
                fire_write(p * _NSLOT + s, s)
            return 0

        lax.fori_loop(0, n_chunks // _NSLOT, body, 0)
        for s in range(_NSLOT):
            drain_write(s)

    return gather_k


def kernel(input_ids, table, W, b):
    B, L = input_ids.shape
    n_tokens = B * L
    table2 = _transform_table(table, W, b)
    flat_idx = input_ids.reshape(n_tokens).astype(jnp.int32)
    out = _make_gather(n_tokens)(table2, flat_idx)
    return out.reshape(B, L, _HID)
